# baseline (speedup 1.0000x reference)
import functools

import jax
import jax.numpy as jnp
from jax.experimental import pallas as pl

_LANES = 128
_ROWS_PER_BLOCK = 782


def _acos(x):
    t = jnp.abs(x)
    r = jnp.sqrt(jnp.maximum(1.0 - t, 0.0)) * (
        1.5707288 + t * (-0.2121144 + t * (0.0742610 + t * (-0.0187293)))
    )
    return jnp.where(x < 0.0, jnp.float32(3.14159265358979) - r, r)


def _finish_body(z0, z1, z2, y0, y1, y2, m, loss_ref, ang_ref):
    normz = jnp.sqrt(z0[...] ** 2 + z1[...] ** 2 + z2[...] ** 2)
    norma = jnp.sqrt(y0[...] ** 2 + y1[...] ** 2 + y2[...] ** 2)
    dot = z0[...] * y0[...] + z1[...] * y1[...] + z2[...] * y2[...]
    cos = dot / (norma * normz)
    cosc = jnp.clip(cos, -1.0, 1.0)
    ang = _acos(cosc) * jnp.float32(57.29577951308232)
    msk = m[...]
    cnt = jnp.sum(msk)
    loss = jnp.sum(jnp.where(msk > 0.0, 1.0 - jnp.abs(cosc), 0.0)) / cnt
    angs = jnp.sum(jnp.where(msk > 0.0, ang, 0.0)) / cnt
    loss_ref[...] = jnp.reshape(loss, (1, 1))
    ang_ref[...] = jnp.reshape(angs, (1, 1))


def _finish(Z, y, mask):
    n = Z.shape[0]
    pad = _ROWS_PER_BLOCK * _LANES - n
    def prep(v):
        v = jnp.pad(v, (0, pad))
        return v.reshape(_ROWS_PER_BLOCK, _LANES)
    args = [prep(Z[:, i]) for i in range(3)] + [prep(y[:, i].astype(jnp.float32)) for i in range(3)]
    args.append(prep(mask.astype(jnp.float32)))
    loss, ang = pl.pallas_call(
        _finish_body,
        out_shape=(
            jax.ShapeDtypeStruct((1, 1), jnp.float32),
            jax.ShapeDtypeStruct((1, 1), jnp.float32),
        ),
    )(*args)
    return loss[0, 0], ang[0, 0]


def kernel(out_scalar_field, x, y, edge_index, mask):
    src = edge_index[0]
    dst = edge_index[1]
    a = jnp.minimum(src, dst)
    b = jnp.maximum(src, dst)
    order = jnp.lexsort((b, a))
    ua = a[order]
    ub = b[order]
    first = jnp.concatenate(
        [jnp.ones((1,), dtype=jnp.bool_), (ua[1:] != ua[:-1]) | (ub[1:] != ub[:-1])]
    )
    S = out_scalar_field
    n = S.shape[0]
    dx = x[ub] - x[ua]
    ds = S[ub] - S[ua]
    contrib = jnp.where(first[:, None], dx * ds[:, None], jnp.float32(0.0))
    Z = jnp.zeros((n, 3), dtype=jnp.float32).at[ua].add(contrib).at[ub].add(contrib)
    return _finish(Z, y, mask)


# speedup vs baseline: 9.6751x; 9.6751x over previous
import numpy as np

import jax
import jax.numpy as jnp
from jax import lax
from jax.experimental import pallas as pl
from jax.experimental.pallas import tpu as pltpu
from jax.experimental.pallas import tpu_sc as plsc

_LANES = 128
_ROWS_PER_BLOCK = 782

_H = 1 << 18
_CH = 128
_NSUB = _CH // 16


def _minmax_body(ei_ref, a_ref, b_ref):
    e = ei_ref[...]
    a_ref[...] = jnp.minimum(e[0], e[1])
    b_ref[...] = jnp.maximum(e[0], e[1])


def _sc_dedup_call(a_e, b_e, px0, px1, px2, pss, zz):
    E = a_e.shape[0]
    N = px0.shape[0]
    EW = E // 16
    RZ = N // 16
    nchunks_b = -(-EW // _CH)
    IDCAP = 56320

    mesh = plsc.VectorSubcoreMesh(core_axis_name="c", subcore_axis_name="s")

    c_h1a = np.uint32(2654435761)
    c_h1b = np.uint32(2246822519)
    c_h2a = np.uint32(3266489917)
    c_h2b = np.uint32(668265263)
    c_pa = np.uint32(0x7FEB352D)
    c_pb = np.uint32(0x846CA68B)
    c_msk = np.uint32(_H - 1)

    def body(a_hbm, b_hbm, hx0, hx1, hx2, hss, zz_hbm, out_hbm,
             tab, ps0, ps1, ps2, ps3, zs0, zs1, zs2,
             ids, abuf, bbuf, hbuf, idbuf, wbuf, wabuf, wbbuf,
             pa0, pa1, pa2, pa3, pb0, pb1, pb2, pb3,
             cr0, cr1, cr2, stg,
             sem1, sem2, sem3):
        c = lax.axis_index("c")
        s = lax.axis_index("s")
        iota = lax.iota(jnp.int32, 16)
        one16 = jnp.ones((16,), jnp.int32)
        zero16 = jnp.zeros((16,), jnp.int32)

        def hashes(av, bv):
            au = av.astype(jnp.uint32)
            bu = bv.astype(jnp.uint32)
            h1 = (au * c_h1a) ^ (bu * c_h1b)
            h2 = ((au * c_h2a) ^ (bu * c_h2b)) | np.uint32(1)
            pm = au * c_pa + bu * c_pb
            par = lax.shift_right_logical(pm, np.uint32(31)).astype(jnp.int32)
            return h1, h2, par

        sl = pl.ds(s * RZ, RZ)
        for hsrc, sdst in ((hx0, ps0), (hx1, ps1), (hx2, ps2), (hss, ps3)):
            pltpu.sync_copy(hsrc.at[sl], stg)
            pltpu.sync_copy(stg, sdst.at[sl])
        pltpu.sync_copy(zz_hbm.at[sl], stg)
        pltpu.sync_copy(stg, zs0.at[sl])
        pltpu.sync_copy(stg, zs1.at[sl])
        pltpu.sync_copy(stg, zs2.at[sl])

        base0 = s * EW

        def b_chunk(i, off):
            cb = base0 + i * _CH
            da = pltpu.async_copy(a_hbm.at[pl.ds(cb, _CH)], abuf, sem1)
            db = pltpu.async_copy(b_hbm.at[pl.ds(cb, _CH)], bbuf, sem1)
            da.wait()
            db.wait()
            rem = EW - i * _CH
            for j in range(_NSUB):
                av = abuf[pl.ds(16 * j, 16)]
                bv = bbuf[pl.ds(16 * j, 16)]
                h1, h2, par = hashes(av, bv)
                valid = (iota + (16 * j)) < rem
                active = valid & (par == c)
                h = (h1 & c_msk).astype(jnp.int32)
                hbuf[pl.ds(16 * j, 16)] = jnp.where(active, h, _H + s)
                idv = cb + 16 * j + iota
                idbuf[pl.ds(16 * j, 16)] = jnp.where(active, idv, 0)
                offc = jnp.minimum(off, IDCAP - 144)
                mi = jnp.where(active, one16, zero16)
                excl = plsc.cumsum(mi) - mi
                tgt = jnp.where(active, offc + excl, IDCAP - 16 + iota)
                plsc.store_scatter(ids, [tgt], idv)
                off = jnp.minimum(off + jnp.sum(mi), IDCAP - 144)
            pltpu.sync_copy(idbuf, tab.at[hbuf])
            return off

        n0 = lax.fori_loop(0, nchunks_b, b_chunk, jnp.int32(0))
        for k in range(8):
            ids[pl.ds(n0 + 16 * k, 16)] = zero16

        def scatter_phase(r, n):
            ru = r.astype(jnp.uint32)
            nch = (n + (_CH - 1)) // _CH

            def chunk(i, _):
                ib = i * _CH
                da = pltpu.async_copy(a_hbm.at[ids.at[pl.ds(ib, _CH)]], abuf, sem1)
                db = pltpu.async_copy(b_hbm.at[ids.at[pl.ds(ib, _CH)]], bbuf, sem1)
                da.wait()
                db.wait()
                rem = n - ib
                for j in range(_NSUB):
                    av = abuf[pl.ds(16 * j, 16)]
                    bv = bbuf[pl.ds(16 * j, 16)]
                    h1, h2, _ = hashes(av, bv)
                    valid = (iota + (16 * j)) < rem
                    h = ((h1 + ru * h2) & c_msk).astype(jnp.int32)
                    hbuf[pl.ds(16 * j, 16)] = jnp.where(valid, h, _H + s)
                    idv = ids[pl.ds(ib + 16 * j, 16)]
                    idbuf[pl.ds(16 * j, 16)] = jnp.where(valid, idv, 0)
                pltpu.sync_copy(idbuf, tab.at[hbuf])
                return 0

            lax.fori_loop(0, nch, chunk, 0)

        def check_phase(r, n):
            ru = r.astype(jnp.uint32)
            nch = (n + (_CH - 1)) // _CH

            def chunk(i, woff):
                ib = i * _CH
                da = pltpu.async_copy(a_hbm.at[ids.at[pl.ds(ib, _CH)]], abuf, sem1)
                db = pltpu.async_copy(b_hbm.at[ids.at[pl.ds(ib, _CH)]], bbuf, sem1)
                da.wait()
                db.wait()
                dg = [
                    pltpu.async_copy(ps0.at[abuf], pa0, sem2),
                    pltpu.async_copy(ps1.at[abuf], pa1, sem2),
                    pltpu.async_copy(ps2.at[abuf], pa2, sem2),
                    pltpu.async_copy(ps3.at[abuf], pa3, sem2),
                    pltpu.async_copy(ps0.at[bbuf], pb0, sem2),
                    pltpu.async_copy(ps1.at[bbuf], pb1, sem2),
                    pltpu.async_copy(ps2.at[bbuf], pb2, sem2),
                    pltpu.async_copy(ps3.at[bbuf], pb3, sem2),
                ]
                rem = n - ib
                for j in range(_NSUB):
                    av = abuf[pl.ds(16 * j, 16)]
                    bv = bbuf[pl.ds(16 * j, 16)]
                    h1, h2, _ = hashes(av, bv)
                    valid = (iota + (16 * j)) < rem
                    h = ((h1 + ru * h2) & c_msk).astype(jnp.int32)
                    hbuf[pl.ds(16 * j, 16)] = jnp.where(valid, h, _H + s)
                pltpu.sync_copy(tab.at[hbuf], wbuf)
                dwa = pltpu.async_copy(a_hbm.at[wbuf], wabuf, sem3)
                dwb = pltpu.async_copy(b_hbm.at[wbuf], wbbuf, sem3)
                for d in dg:
                    d.wait()
                dwa.wait()
                dwb.wait()
                for j in range(_NSUB):
                    jj = pl.ds(16 * j, 16)
                    av = abuf[jj]
                    bv = bbuf[jj]
                    wv = wbuf[jj]
                    wav = wabuf[jj]
                    wbv = wbbuf[jj]
                    idv = ids[pl.ds(ib + 16 * j, 16)]
                    valid = (iota + (16 * j)) < (n - ib)
                    selfw = (wv == idv) & valid
                    pmatch = (wav == av) & (wbv == bv)
                    surv = valid & jnp.logical_not(selfw) & jnp.logical_not(pmatch)
                    keepf = jnp.where(selfw, jnp.float32(1.0), jnp.float32(0.0))
                    dsv = (pb3[jj] - pa3[jj]) * keepf
                    cr0[jj] = (pb0[jj] - pa0[jj]) * dsv
                    cr1[jj] = (pb1[jj] - pa1[jj]) * dsv
                    cr2[jj] = (pb2[jj] - pa2[jj]) * dsv
                    woffc = jnp.minimum(woff, IDCAP - 144)
                    mi = jnp.where(surv, one16, zero16)
                    excl = plsc.cumsum(mi) - mi
                    tgt = jnp.where(surv, woffc + excl, IDCAP - 16 + iota)
                    plsc.store_scatter(ids, [tgt], idv)
                    woff = jnp.minimum(woff + jnp.sum(mi), IDCAP - 144)
                pltpu.sync_copy(cr0, zs0.at[abuf], add=True)
                pltpu.sync_copy(cr1, zs1.at[abuf], add=True)
                pltpu.sync_copy(cr2, zs2.at[abuf], add=True)
                pltpu.sync_copy(cr0, zs0.at[bbuf], add=True)
                pltpu.sync_copy(cr1, zs1.at[bbuf], add=True)
                pltpu.sync_copy(cr2, zs2.at[bbuf], add=True)
                return woff

            nnew = lax.fori_loop(0, nch, chunk, jnp.int32(0))
            for k in range(8):
                ids[pl.ds(nnew + 16 * k, 16)] = zero16
            return nnew

        plsc.subcore_barrier()
        n1 = check_phase(jnp.int32(0), n0)
        plsc.subcore_barrier()

        def rbody(r, n):
            scatter_phase(r, n)
            plsc.subcore_barrier()
            n2 = check_phase(r, n)
            plsc.subcore_barrier()
            return n2

        lax.fori_loop(1, 13, rbody, n1)

        ob = c * (3 * N) + s * RZ
        for k2, zsrc in enumerate((zs0, zs1, zs2)):
            pltpu.sync_copy(zsrc.at[sl], stg)
            pltpu.sync_copy(stg, out_hbm.at[pl.ds(ob + k2 * N, RZ)])

    f = pl.kernel(
        body,
        out_type=jax.ShapeDtypeStruct((2 * 3 * N,), jnp.float32),
        mesh=mesh,
        compiler_params=pltpu.CompilerParams(needs_layout_passes=False),
        scratch_types=[
            pltpu.VMEM_SHARED((_H + 16,), jnp.int32),
            pltpu.VMEM_SHARED((N,), jnp.float32),
            pltpu.VMEM_SHARED((N,), jnp.float32),
            pltpu.VMEM_SHARED((N,), jnp.float32),
            pltpu.VMEM_SHARED((N,), jnp.float32),
            pltpu.VMEM_SHARED((N,), jnp.float32),
            pltpu.VMEM_SHARED((N,), jnp.float32),
            pltpu.VMEM_SHARED((N,), jnp.float32),
            pltpu.VMEM((IDCAP,), jnp.int32),
            pltpu.VMEM((_CH,), jnp.int32),
            pltpu.VMEM((_CH,), jnp.int32),
            pltpu.VMEM((_CH,), jnp.int32),
            pltpu.VMEM((_CH,), jnp.int32),
            pltpu.VMEM((_CH,), jnp.int32),
            pltpu.VMEM((_CH,), jnp.int32),
            pltpu.VMEM((_CH,), jnp.int32),
            pltpu.VMEM((_CH,), jnp.float32),
            pltpu.VMEM((_CH,), jnp.float32),
            pltpu.VMEM((_CH,), jnp.float32),
            pltpu.VMEM((_CH,), jnp.float32),
            pltpu.VMEM((_CH,), jnp.float32),
            pltpu.VMEM((_CH,), jnp.float32),
            pltpu.VMEM((_CH,), jnp.float32),
            pltpu.VMEM((_CH,), jnp.float32),
            pltpu.VMEM((_CH,), jnp.float32),
            pltpu.VMEM((_CH,), jnp.float32),
            pltpu.VMEM((_CH,), jnp.float32),
            pltpu.VMEM((6256,), jnp.float32),
            pltpu.SemaphoreType.DMA,
            pltpu.SemaphoreType.DMA,
            pltpu.SemaphoreType.DMA,
        ],
    )
    return f(a_e, b_e, px0, px1, px2, pss, zz)


def _acos(x):
    t = jnp.abs(x)
    r = jnp.sqrt(jnp.maximum(1.0 - t, 0.0)) * (
        1.5707288 + t * (-0.2121144 + t * (0.0742610 + t * (-0.0187293)))
    )
    return jnp.where(x < 0.0, jnp.float32(3.14159265358979) - r, r)


def _finish_body(za0, za1, za2, zb0, zb1, zb2, y0, y1, y2, m,
                 loss_ref, ang_ref):
    z0 = za0[...] + zb0[...]
    z1 = za1[...] + zb1[...]
    z2 = za2[...] + zb2[...]
    normz = jnp.sqrt(z0 ** 2 + z1 ** 2 + z2 ** 2)
    norma = jnp.sqrt(y0[...] ** 2 + y1[...] ** 2 + y2[...] ** 2)
    dot = z0 * y0[...] + z1 * y1[...] + z2 * y2[...]
    cos = dot / (norma * normz)
    cosc = jnp.clip(cos, -1.0, 1.0)
    ang = _acos(cosc) * jnp.float32(57.29577951308232)
    msk = m[...]
    cnt = jnp.sum(msk)
    loss = jnp.sum(jnp.where(msk > 0.0, 1.0 - jnp.abs(cosc), 0.0)) / cnt
    angs = jnp.sum(jnp.where(msk > 0.0, ang, 0.0)) / cnt
    loss_ref[...] = jnp.reshape(loss, (1, 1))
    ang_ref[...] = jnp.reshape(angs, (1, 1))


def _finish(zparts, y, mask):
    n = y.shape[0]
    pad = _ROWS_PER_BLOCK * _LANES - n

    def prep(v):
        v = jnp.pad(v, (0, pad))
        return v.reshape(_ROWS_PER_BLOCK, _LANES)

    def prep2(v):
        return v.reshape(_ROWS_PER_BLOCK, _LANES)

    args = [prep2(zparts[0, i]) for i in range(3)]
    args += [prep2(zparts[1, i]) for i in range(3)]
    args += [prep(y[:, i].astype(jnp.float32)) for i in range(3)]
    args.append(prep(mask.astype(jnp.float32)))
    loss, ang = pl.pallas_call(
        _finish_body,
        out_shape=(
            jax.ShapeDtypeStruct((1, 1), jnp.float32),
            jax.ShapeDtypeStruct((1, 1), jnp.float32),
        ),
    )(*args)
    return loss[0, 0], ang[0, 0]


def kernel(out_scalar_field, x, y, edge_index, mask):
    E = edge_index.shape[1]
    ei = edge_index.reshape(2, E // _LANES, _LANES)
    a2, b2 = pl.pallas_call(
        _minmax_body,
        out_shape=(
            jax.ShapeDtypeStruct((E // _LANES, _LANES), jnp.int32),
            jax.ShapeDtypeStruct((E // _LANES, _LANES), jnp.int32),
        ),
    )(ei)
    a = a2.reshape(E)
    b = b2.reshape(E)
    n = out_scalar_field.shape[0]
    npad = _ROWS_PER_BLOCK * _LANES - n
    xf = x.astype(jnp.float32)
    px0 = jnp.pad(xf[:, 0], (0, npad))
    px1 = jnp.pad(xf[:, 1], (0, npad))
    px2 = jnp.pad(xf[:, 2], (0, npad))
    pss = jnp.pad(out_scalar_field.astype(jnp.float32), (0, npad))
    zz = jnp.zeros((n + npad,), jnp.float32)
    zflat = _sc_dedup_call(a, b, px0, px1, px2, pss, zz)
    zparts = zflat.reshape(2, 3, n + npad)
    return _finish(zparts, y, mask)


# speedup vs baseline: 12.7196x; 1.3147x over previous
import numpy as np

import jax
import jax.numpy as jnp
from jax import lax
from jax.experimental import pallas as pl
from jax.experimental.pallas import tpu as pltpu
from jax.experimental.pallas import tpu_sc as plsc

_LANES = 128
_ROWS_PER_BLOCK = 782

_H = 1 << 18
_CH = 256
_NSUB = _CH // 16


def _minmax_body(ei_ref, a_ref, b_ref):
    e = ei_ref[...]
    a_ref[...] = jnp.minimum(e[0], e[1])
    b_ref[...] = jnp.maximum(e[0], e[1])


def _sc_dedup_call(a_e, b_e, px0, px1, px2, pss, zz):
    E = a_e.shape[0]
    N = px0.shape[0]
    EW = E // 16
    RZ = N // 16
    nchunks_b = -(-EW // _CH)
    IDCAP = 56320

    mesh = plsc.VectorSubcoreMesh(core_axis_name="c", subcore_axis_name="s")

    c_h1a = np.uint32(2654435761)
    c_h1b = np.uint32(2246822519)
    c_h2a = np.uint32(3266489917)
    c_h2b = np.uint32(668265263)
    c_pa = np.uint32(0x7FEB352D)
    c_pb = np.uint32(0x846CA68B)
    c_msk = np.uint32(_H - 1)

    def body(a_hbm, b_hbm, hx0, hx1, hx2, hss, zz_hbm, out_hbm,
             tab, ps0, ps1, ps2, ps3, zs0, zs1, zs2,
             ids, abuf, bbuf, hbuf, idbuf, wbuf, wabuf, wbbuf,
             pa0, pa1, pa2, pa3, pb0, pb1, pb2, pb3,
             cr0, cr1, cr2, stg,
             sem1, sem2, sem3):
        c = lax.axis_index("c")
        s = lax.axis_index("s")
        iota = lax.iota(jnp.int32, 16)
        one16 = jnp.ones((16,), jnp.int32)
        zero16 = jnp.zeros((16,), jnp.int32)

        def hashes(av, bv):
            au = av.astype(jnp.uint32)
            bu = bv.astype(jnp.uint32)
            h1 = (au * c_h1a) ^ (bu * c_h1b)
            h2 = ((au * c_h2a) ^ (bu * c_h2b)) | np.uint32(1)
            pm = au * c_pa + bu * c_pb
            par = lax.shift_right_logical(pm, np.uint32(31)).astype(jnp.int32)
            return h1, h2, par

        sl = pl.ds(s * RZ, RZ)
        for hsrc, sdst in ((hx0, ps0), (hx1, ps1), (hx2, ps2), (hss, ps3)):
            pltpu.sync_copy(hsrc.at[sl], stg)
            pltpu.sync_copy(stg, sdst.at[sl])
        pltpu.sync_copy(zz_hbm.at[sl], stg)
        pltpu.sync_copy(stg, zs0.at[sl])
        pltpu.sync_copy(stg, zs1.at[sl])
        pltpu.sync_copy(stg, zs2.at[sl])

        base0 = s * EW

        def b_chunk(i, off):
            cb = base0 + i * _CH
            da = pltpu.async_copy(a_hbm.at[pl.ds(cb, _CH)], abuf, sem1)
            db = pltpu.async_copy(b_hbm.at[pl.ds(cb, _CH)], bbuf, sem1)
            da.wait()
            db.wait()
            rem = EW - i * _CH
            for j in range(_NSUB):
                av = abuf[pl.ds(16 * j, 16)]
                bv = bbuf[pl.ds(16 * j, 16)]
                h1, h2, par = hashes(av, bv)
                valid = (iota + (16 * j)) < rem
                active = valid & (par == c)
                h = (h1 & c_msk).astype(jnp.int32)
                hbuf[pl.ds(16 * j, 16)] = jnp.where(active, h, _H + s)
                idv = cb + 16 * j + iota
                idbuf[pl.ds(16 * j, 16)] = jnp.where(active, idv, 0)
                offc = jnp.minimum(off, IDCAP - 144)
                mi = jnp.where(active, one16, zero16)
                excl = plsc.cumsum(mi) - mi
                tgt = jnp.where(active, offc + excl, IDCAP - 16 + iota)
                plsc.store_scatter(ids, [tgt], idv)
                off = jnp.minimum(off + jnp.sum(mi), IDCAP - 144)
            pltpu.sync_copy(idbuf, tab.at[hbuf])
            return off

        n0 = lax.fori_loop(0, nchunks_b, b_chunk, jnp.int32(0))
        for k in range(8):
            ids[pl.ds(n0 + 16 * k, 16)] = zero16

        def scatter_phase(r, n):
            ru = r.astype(jnp.uint32)
            nch = (n + (_CH - 1)) // _CH

            def chunk(i, _):
                ib = i * _CH
                da = pltpu.async_copy(a_hbm.at[ids.at[pl.ds(ib, _CH)]], abuf, sem1)
                db = pltpu.async_copy(b_hbm.at[ids.at[pl.ds(ib, _CH)]], bbuf, sem1)
                da.wait()
                db.wait()
                rem = n - ib
                for j in range(_NSUB):
                    av = abuf[pl.ds(16 * j, 16)]
                    bv = bbuf[pl.ds(16 * j, 16)]
                    h1, h2, _ = hashes(av, bv)
                    valid = (iota + (16 * j)) < rem
                    h = ((h1 + ru * h2) & c_msk).astype(jnp.int32)
                    hbuf[pl.ds(16 * j, 16)] = jnp.where(valid, h, _H + s)
                    idv = ids[pl.ds(ib + 16 * j, 16)]
                    idbuf[pl.ds(16 * j, 16)] = jnp.where(valid, idv, 0)
                pltpu.sync_copy(idbuf, tab.at[hbuf])
                return 0

            lax.fori_loop(0, nch, chunk, 0)

        def check_phase(r, n):
            ru = r.astype(jnp.uint32)
            nch = (n + (_CH - 1)) // _CH

            def chunk(i, woff):
                ib = i * _CH
                da = pltpu.async_copy(a_hbm.at[ids.at[pl.ds(ib, _CH)]], abuf, sem1)
                db = pltpu.async_copy(b_hbm.at[ids.at[pl.ds(ib, _CH)]], bbuf, sem1)
                da.wait()
                db.wait()
                dg = [
                    pltpu.async_copy(ps0.at[abuf], pa0, sem2),
                    pltpu.async_copy(ps1.at[abuf], pa1, sem2),
                    pltpu.async_copy(ps2.at[abuf], pa2, sem2),
                    pltpu.async_copy(ps3.at[abuf], pa3, sem2),
                    pltpu.async_copy(ps0.at[bbuf], pb0, sem2),
                    pltpu.async_copy(ps1.at[bbuf], pb1, sem2),
                    pltpu.async_copy(ps2.at[bbuf], pb2, sem2),
                    pltpu.async_copy(ps3.at[bbuf], pb3, sem2),
                ]
                rem = n - ib
                for j in range(_NSUB):
                    av = abuf[pl.ds(16 * j, 16)]
                    bv = bbuf[pl.ds(16 * j, 16)]
                    h1, h2, _ = hashes(av, bv)
                    valid = (iota + (16 * j)) < rem
                    h = ((h1 + ru * h2) & c_msk).astype(jnp.int32)
                    hbuf[pl.ds(16 * j, 16)] = jnp.where(valid, h, _H + s)
                pltpu.sync_copy(tab.at[hbuf], wbuf)
                dwa = pltpu.async_copy(a_hbm.at[wbuf], wabuf, sem3)
                dwb = pltpu.async_copy(b_hbm.at[wbuf], wbbuf, sem3)
                for d in dg:
                    d.wait()
                dwa.wait()
                dwb.wait()
                for j in range(_NSUB):
                    jj = pl.ds(16 * j, 16)
                    av = abuf[jj]
                    bv = bbuf[jj]
                    wv = wbuf[jj]
                    wav = wabuf[jj]
                    wbv = wbbuf[jj]
                    idv = ids[pl.ds(ib + 16 * j, 16)]
                    valid = (iota + (16 * j)) < (n - ib)
                    selfw = (wv == idv) & valid
                    pmatch = (wav == av) & (wbv == bv)
                    surv = valid & jnp.logical_not(selfw) & jnp.logical_not(pmatch)
                    keepf = jnp.where(selfw, jnp.float32(1.0), jnp.float32(0.0))
                    dsv = (pb3[jj] - pa3[jj]) * keepf
                    cr0[jj] = (pb0[jj] - pa0[jj]) * dsv
                    cr1[jj] = (pb1[jj] - pa1[jj]) * dsv
                    cr2[jj] = (pb2[jj] - pa2[jj]) * dsv
                    woffc = jnp.minimum(woff, IDCAP - 144)
                    mi = jnp.where(surv, one16, zero16)
                    excl = plsc.cumsum(mi) - mi
                    tgt = jnp.where(surv, woffc + excl, IDCAP - 16 + iota)
                    plsc.store_scatter(ids, [tgt], idv)
                    woff = jnp.minimum(woff + jnp.sum(mi), IDCAP - 144)
                pltpu.sync_copy(cr0, zs0.at[abuf], add=True)
                pltpu.sync_copy(cr1, zs1.at[abuf], add=True)
                pltpu.sync_copy(cr2, zs2.at[abuf], add=True)
                pltpu.sync_copy(cr0, zs0.at[bbuf], add=True)
                pltpu.sync_copy(cr1, zs1.at[bbuf], add=True)
                pltpu.sync_copy(cr2, zs2.at[bbuf], add=True)
                return woff

            nnew = lax.fori_loop(0, nch, chunk, jnp.int32(0))
            for k in range(8):
                ids[pl.ds(nnew + 16 * k, 16)] = zero16
            return nnew

        plsc.subcore_barrier()
        n1 = check_phase(jnp.int32(0), n0)
        plsc.subcore_barrier()

        def rbody(r, n):
            scatter_phase(r, n)
            plsc.subcore_barrier()
            n2 = check_phase(r, n)
            plsc.subcore_barrier()
            return n2

        lax.fori_loop(1, 13, rbody, n1)

        ob = c * (3 * N) + s * RZ
        for k2, zsrc in enumerate((zs0, zs1, zs2)):
            pltpu.sync_copy(zsrc.at[sl], stg)
            pltpu.sync_copy(stg, out_hbm.at[pl.ds(ob + k2 * N, RZ)])

    f = pl.kernel(
        body,
        out_type=jax.ShapeDtypeStruct((2 * 3 * N,), jnp.float32),
        mesh=mesh,
        compiler_params=pltpu.CompilerParams(needs_layout_passes=False),
        scratch_types=[
            pltpu.VMEM_SHARED((_H + 16,), jnp.int32),
            pltpu.VMEM_SHARED((N,), jnp.float32),
            pltpu.VMEM_SHARED((N,), jnp.float32),
            pltpu.VMEM_SHARED((N,), jnp.float32),
            pltpu.VMEM_SHARED((N,), jnp.float32),
            pltpu.VMEM_SHARED((N,), jnp.float32),
            pltpu.VMEM_SHARED((N,), jnp.float32),
            pltpu.VMEM_SHARED((N,), jnp.float32),
            pltpu.VMEM((IDCAP,), jnp.int32),
            pltpu.VMEM((_CH,), jnp.int32),
            pltpu.VMEM((_CH,), jnp.int32),
            pltpu.VMEM((_CH,), jnp.int32),
            pltpu.VMEM((_CH,), jnp.int32),
            pltpu.VMEM((_CH,), jnp.int32),
            pltpu.VMEM((_CH,), jnp.int32),
            pltpu.VMEM((_CH,), jnp.int32),
            pltpu.VMEM((_CH,), jnp.float32),
            pltpu.VMEM((_CH,), jnp.float32),
            pltpu.VMEM((_CH,), jnp.float32),
            pltpu.VMEM((_CH,), jnp.float32),
            pltpu.VMEM((_CH,), jnp.float32),
            pltpu.VMEM((_CH,), jnp.float32),
            pltpu.VMEM((_CH,), jnp.float32),
            pltpu.VMEM((_CH,), jnp.float32),
            pltpu.VMEM((_CH,), jnp.float32),
            pltpu.VMEM((_CH,), jnp.float32),
            pltpu.VMEM((_CH,), jnp.float32),
            pltpu.VMEM((6256,), jnp.float32),
            pltpu.SemaphoreType.DMA,
            pltpu.SemaphoreType.DMA,
            pltpu.SemaphoreType.DMA,
        ],
    )
    return f(a_e, b_e, px0, px1, px2, pss, zz)


def _acos(x):
    t = jnp.abs(x)
    r = jnp.sqrt(jnp.maximum(1.0 - t, 0.0)) * (
        1.5707288 + t * (-0.2121144 + t * (0.0742610 + t * (-0.0187293)))
    )
    return jnp.where(x < 0.0, jnp.float32(3.14159265358979) - r, r)


def _finish_body(za0, za1, za2, zb0, zb1, zb2, y0, y1, y2, m,
                 loss_ref, ang_ref):
    z0 = za0[...] + zb0[...]
    z1 = za1[...] + zb1[...]
    z2 = za2[...] + zb2[...]
    normz = jnp.sqrt(z0 ** 2 + z1 ** 2 + z2 ** 2)
    norma = jnp.sqrt(y0[...] ** 2 + y1[...] ** 2 + y2[...] ** 2)
    dot = z0 * y0[...] + z1 * y1[...] + z2 * y2[...]
    cos = dot / (norma * normz)
    cosc = jnp.clip(cos, -1.0, 1.0)
    ang = _acos(cosc) * jnp.float32(57.29577951308232)
    msk = m[...]
    cnt = jnp.sum(msk)
    loss = jnp.sum(jnp.where(msk > 0.0, 1.0 - jnp.abs(cosc), 0.0)) / cnt
    angs = jnp.sum(jnp.where(msk > 0.0, ang, 0.0)) / cnt
    loss_ref[...] = jnp.reshape(loss, (1, 1))
    ang_ref[...] = jnp.reshape(angs, (1, 1))


def _finish(zparts, y, mask):
    n = y.shape[0]
    pad = _ROWS_PER_BLOCK * _LANES - n

    def prep(v):
        v = jnp.pad(v, (0, pad))
        return v.reshape(_ROWS_PER_BLOCK, _LANES)

    def prep2(v):
        return v.reshape(_ROWS_PER_BLOCK, _LANES)

    args = [prep2(zparts[0, i]) for i in range(3)]
    args += [prep2(zparts[1, i]) for i in range(3)]
    args += [prep(y[:, i].astype(jnp.float32)) for i in range(3)]
    args.append(prep(mask.astype(jnp.float32)))
    loss, ang = pl.pallas_call(
        _finish_body,
        out_shape=(
            jax.ShapeDtypeStruct((1, 1), jnp.float32),
            jax.ShapeDtypeStruct((1, 1), jnp.float32),
        ),
    )(*args)
    return loss[0, 0], ang[0, 0]


def kernel(out_scalar_field, x, y, edge_index, mask):
    E = edge_index.shape[1]
    ei = edge_index.reshape(2, E // _LANES, _LANES)
    a2, b2 = pl.pallas_call(
        _minmax_body,
        out_shape=(
            jax.ShapeDtypeStruct((E // _LANES, _LANES), jnp.int32),
            jax.ShapeDtypeStruct((E // _LANES, _LANES), jnp.int32),
        ),
    )(ei)
    a = a2.reshape(E)
    b = b2.reshape(E)
    n = out_scalar_field.shape[0]
    npad = _ROWS_PER_BLOCK * _LANES - n
    xf = x.astype(jnp.float32)
    px0 = jnp.pad(xf[:, 0], (0, npad))
    px1 = jnp.pad(xf[:, 1], (0, npad))
    px2 = jnp.pad(xf[:, 2], (0, npad))
    pss = jnp.pad(out_scalar_field.astype(jnp.float32), (0, npad))
    zz = jnp.zeros((n + npad,), jnp.float32)
    zflat = _sc_dedup_call(a, b, px0, px1, px2, pss, zz)
    zparts = zflat.reshape(2, 3, n + npad)
    return _finish(zparts, y, mask)


# speedup vs baseline: 14.7309x; 1.1581x over previous
import numpy as np

import jax
import jax.numpy as jnp
from jax import lax
from jax.experimental import pallas as pl
from jax.experimental.pallas import tpu as pltpu
from jax.experimental.pallas import tpu_sc as plsc

_LANES = 128
_ROWS_PER_BLOCK = 782

_H = 1 << 18
_CH = 512
_NSUB = _CH // 16


def _minmax_body(ei_ref, a_ref, b_ref):
    e = ei_ref[...]
    a_ref[...] = jnp.minimum(e[0], e[1])
    b_ref[...] = jnp.maximum(e[0], e[1])


def _sc_dedup_call(a_e, b_e, px0, px1, px2, pss, zz):
    E = a_e.shape[0]
    N = px0.shape[0]
    EW = E // 16
    RZ = N // 16
    nchunks_b = -(-EW // _CH)
    IDCAP = 52480

    mesh = plsc.VectorSubcoreMesh(core_axis_name="c", subcore_axis_name="s")

    c_h1a = np.uint32(2654435761)
    c_h1b = np.uint32(2246822519)
    c_h2a = np.uint32(3266489917)
    c_h2b = np.uint32(668265263)
    c_pa = np.uint32(0x7FEB352D)
    c_pb = np.uint32(0x846CA68B)
    c_msk = np.uint32(_H - 1)

    def body(a_hbm, b_hbm, hx0, hx1, hx2, hss, zz_hbm, out_hbm,
             tab, ps0, ps1, ps2, ps3, zs0, zs1, zs2,
             ids, abuf, bbuf, hbuf, idbuf, wbuf, wabuf, wbbuf,
             pa0, pa1, pa2, pa3, pb0, pb1, pb2, pb3,
             cr0, cr1, cr2,
             sem1, sem2, sem3):
        c = lax.axis_index("c")
        s = lax.axis_index("s")
        iota = lax.iota(jnp.int32, 16)
        one16 = jnp.ones((16,), jnp.int32)
        zero16 = jnp.zeros((16,), jnp.int32)

        def hashes(av, bv):
            au = av.astype(jnp.uint32)
            bu = bv.astype(jnp.uint32)
            h1 = (au * c_h1a) ^ (bu * c_h1b)
            h2 = ((au * c_h2a) ^ (bu * c_h2b)) | np.uint32(1)
            pm = au * c_pa + bu * c_pb
            par = lax.shift_right_logical(pm, np.uint32(31)).astype(jnp.int32)
            return h1, h2, par

        r0 = s * RZ
        nfull = RZ // _CH
        rtail = RZ - nfull * _CH
        segs = [(k * _CH, _CH) for k in range(nfull)]
        if rtail:
            segs.append((nfull * _CH, rtail))
        for hsrc, sdst in ((hx0, ps0), (hx1, ps1), (hx2, ps2), (hss, ps3),
                           (zz_hbm, zs0), (zz_hbm, zs1), (zz_hbm, zs2)):
            for o, ln in segs:
                pltpu.sync_copy(hsrc.at[pl.ds(r0 + o, ln)], pa0.at[pl.ds(0, ln)])
                pltpu.sync_copy(pa0.at[pl.ds(0, ln)], sdst.at[pl.ds(r0 + o, ln)])

        base0 = s * EW

        def b_chunk(i, off):
            cb = base0 + i * _CH
            da = pltpu.async_copy(a_hbm.at[pl.ds(cb, _CH)], abuf, sem1)
            db = pltpu.async_copy(b_hbm.at[pl.ds(cb, _CH)], bbuf, sem1)
            da.wait()
            db.wait()
            rem = EW - i * _CH
            for j in range(_NSUB):
                av = abuf[pl.ds(16 * j, 16)]
                bv = bbuf[pl.ds(16 * j, 16)]
                h1, h2, par = hashes(av, bv)
                valid = (iota + (16 * j)) < rem
                active = valid & (par == c)
                h = (h1 & c_msk).astype(jnp.int32)
                hbuf[pl.ds(16 * j, 16)] = jnp.where(active, h, _H + s)
                idv = cb + 16 * j + iota
                idbuf[pl.ds(16 * j, 16)] = jnp.where(active, idv, 0)
                offc = jnp.minimum(off, IDCAP - 144)
                mi = jnp.where(active, one16, zero16)
                excl = plsc.cumsum(mi) - mi
                tgt = jnp.where(active, offc + excl, IDCAP - 16 + iota)
                plsc.store_scatter(ids, [tgt], idv)
                off = jnp.minimum(off + jnp.sum(mi), IDCAP - 144)
            pltpu.sync_copy(idbuf, tab.at[hbuf])
            return off

        n0 = lax.fori_loop(0, nchunks_b, b_chunk, jnp.int32(0))
        for k in range(8):
            ids[pl.ds(n0 + 16 * k, 16)] = zero16

        def scatter_phase(r, n):
            ru = r.astype(jnp.uint32)
            nch = (n + (_CH - 1)) // _CH

            def chunk(i, _):
                ib = i * _CH
                da = pltpu.async_copy(a_hbm.at[ids.at[pl.ds(ib, _CH)]], abuf, sem1)
                db = pltpu.async_copy(b_hbm.at[ids.at[pl.ds(ib, _CH)]], bbuf, sem1)
                da.wait()
                db.wait()
                rem = n - ib
                for j in range(_NSUB):
                    av = abuf[pl.ds(16 * j, 16)]
                    bv = bbuf[pl.ds(16 * j, 16)]
                    h1, h2, _ = hashes(av, bv)
                    valid = (iota + (16 * j)) < rem
                    h = ((h1 + ru * h2) & c_msk).astype(jnp.int32)
                    hbuf[pl.ds(16 * j, 16)] = jnp.where(valid, h, _H + s)
                    idv = ids[pl.ds(ib + 16 * j, 16)]
                    idbuf[pl.ds(16 * j, 16)] = jnp.where(valid, idv, 0)
                pltpu.sync_copy(idbuf, tab.at[hbuf])
                return 0

            lax.fori_loop(0, nch, chunk, 0)

        def check_phase(r, n):
            ru = r.astype(jnp.uint32)
            nch = (n + (_CH - 1)) // _CH

            def chunk(i, woff):
                ib = i * _CH
                da = pltpu.async_copy(a_hbm.at[ids.at[pl.ds(ib, _CH)]], abuf, sem1)
                db = pltpu.async_copy(b_hbm.at[ids.at[pl.ds(ib, _CH)]], bbuf, sem1)
                da.wait()
                db.wait()
                dg = [
                    pltpu.async_copy(ps0.at[abuf], pa0, sem2),
                    pltpu.async_copy(ps1.at[abuf], pa1, sem2),
                    pltpu.async_copy(ps2.at[abuf], pa2, sem2),
                    pltpu.async_copy(ps3.at[abuf], pa3, sem2),
                    pltpu.async_copy(ps0.at[bbuf], pb0, sem2),
                    pltpu.async_copy(ps1.at[bbuf], pb1, sem2),
                    pltpu.async_copy(ps2.at[bbuf], pb2, sem2),
                    pltpu.async_copy(ps3.at[bbuf], pb3, sem2),
                ]
                rem = n - ib
                for j in range(_NSUB):
                    av = abuf[pl.ds(16 * j, 16)]
                    bv = bbuf[pl.ds(16 * j, 16)]
                    h1, h2, _ = hashes(av, bv)
                    valid = (iota + (16 * j)) < rem
                    h = ((h1 + ru * h2) & c_msk).astype(jnp.int32)
                    hbuf[pl.ds(16 * j, 16)] = jnp.where(valid, h, _H + s)
                pltpu.sync_copy(tab.at[hbuf], wbuf)
                dwa = pltpu.async_copy(a_hbm.at[wbuf], wabuf, sem3)
                dwb = pltpu.async_copy(b_hbm.at[wbuf], wbbuf, sem3)
                for d in dg:
                    d.wait()
                dwa.wait()
                dwb.wait()
                for j in range(_NSUB):
                    jj = pl.ds(16 * j, 16)
                    av = abuf[jj]
                    bv = bbuf[jj]
                    wv = wbuf[jj]
                    wav = wabuf[jj]
                    wbv = wbbuf[jj]
                    idv = ids[pl.ds(ib + 16 * j, 16)]
                    valid = (iota + (16 * j)) < (n - ib)
                    selfw = (wv == idv) & valid
                    pmatch = (wav == av) & (wbv == bv)
                    surv = valid & jnp.logical_not(selfw) & jnp.logical_not(pmatch)
                    keepf = jnp.where(selfw, jnp.float32(1.0), jnp.float32(0.0))
                    dsv = (pb3[jj] - pa3[jj]) * keepf
                    cr0[jj] = (pb0[jj] - pa0[jj]) * dsv
                    cr1[jj] = (pb1[jj] - pa1[jj]) * dsv
                    cr2[jj] = (pb2[jj] - pa2[jj]) * dsv
                    woffc = jnp.minimum(woff, IDCAP - 144)
                    mi = jnp.where(surv, one16, zero16)
                    excl = plsc.cumsum(mi) - mi
                    tgt = jnp.where(surv, woffc + excl, IDCAP - 16 + iota)
                    plsc.store_scatter(ids, [tgt], idv)
                    woff = jnp.minimum(woff + jnp.sum(mi), IDCAP - 144)
                pltpu.sync_copy(cr0, zs0.at[abuf], add=True)
                pltpu.sync_copy(cr1, zs1.at[abuf], add=True)
                pltpu.sync_copy(cr2, zs2.at[abuf], add=True)
                pltpu.sync_copy(cr0, zs0.at[bbuf], add=True)
                pltpu.sync_copy(cr1, zs1.at[bbuf], add=True)
                pltpu.sync_copy(cr2, zs2.at[bbuf], add=True)
                return woff

            nnew = lax.fori_loop(0, nch, chunk, jnp.int32(0))
            for k in range(8):
                ids[pl.ds(nnew + 16 * k, 16)] = zero16
            return nnew

        plsc.subcore_barrier()
        n1 = check_phase(jnp.int32(0), n0)
        plsc.subcore_barrier()

        def rbody(r, n):
            scatter_phase(r, n)
            plsc.subcore_barrier()
            n2 = check_phase(r, n)
            plsc.subcore_barrier()
            return n2

        lax.fori_loop(1, 13, rbody, n1)

        ob = c * (3 * N) + s * RZ
        for k2, zsrc in enumerate((zs0, zs1, zs2)):
            for o, ln in segs:
                pltpu.sync_copy(zsrc.at[pl.ds(r0 + o, ln)], pa0.at[pl.ds(0, ln)])
                pltpu.sync_copy(pa0.at[pl.ds(0, ln)],
                                out_hbm.at[pl.ds(ob + k2 * N + o, ln)])

    f = pl.kernel(
        body,
        out_type=jax.ShapeDtypeStruct((2 * 3 * N,), jnp.float32),
        mesh=mesh,
        compiler_params=pltpu.CompilerParams(needs_layout_passes=False),
        scratch_types=[
            pltpu.VMEM_SHARED((_H + 16,), jnp.int32),
            pltpu.VMEM_SHARED((N,), jnp.float32),
            pltpu.VMEM_SHARED((N,), jnp.float32),
            pltpu.VMEM_SHARED((N,), jnp.float32),
            pltpu.VMEM_SHARED((N,), jnp.float32),
            pltpu.VMEM_SHARED((N,), jnp.float32),
            pltpu.VMEM_SHARED((N,), jnp.float32),
            pltpu.VMEM_SHARED((N,), jnp.float32),
            pltpu.VMEM((IDCAP,), jnp.int32),
            pltpu.VMEM((_CH,), jnp.int32),
            pltpu.VMEM((_CH,), jnp.int32),
            pltpu.VMEM((_CH,), jnp.int32),
            pltpu.VMEM((_CH,), jnp.int32),
            pltpu.VMEM((_CH,), jnp.int32),
            pltpu.VMEM((_CH,), jnp.int32),
            pltpu.VMEM((_CH,), jnp.int32),
            pltpu.VMEM((_CH,), jnp.float32),
            pltpu.VMEM((_CH,), jnp.float32),
            pltpu.VMEM((_CH,), jnp.float32),
            pltpu.VMEM((_CH,), jnp.float32),
            pltpu.VMEM((_CH,), jnp.float32),
            pltpu.VMEM((_CH,), jnp.float32),
            pltpu.VMEM((_CH,), jnp.float32),
            pltpu.VMEM((_CH,), jnp.float32),
            pltpu.VMEM((_CH,), jnp.float32),
            pltpu.VMEM((_CH,), jnp.float32),
            pltpu.VMEM((_CH,), jnp.float32),
            pltpu.SemaphoreType.DMA,
            pltpu.SemaphoreType.DMA,
            pltpu.SemaphoreType.DMA,
        ],
    )
    return f(a_e, b_e, px0, px1, px2, pss, zz)


def _acos(x):
    t = jnp.abs(x)
    r = jnp.sqrt(jnp.maximum(1.0 - t, 0.0)) * (
        1.5707288 + t * (-0.2121144 + t * (0.0742610 + t * (-0.0187293)))
    )
    return jnp.where(x < 0.0, jnp.float32(3.14159265358979) - r, r)


def _finish_body(za0, za1, za2, zb0, zb1, zb2, y0, y1, y2, m,
                 loss_ref, ang_ref):
    z0 = za0[...] + zb0[...]
    z1 = za1[...] + zb1[...]
    z2 = za2[...] + zb2[...]
    normz = jnp.sqrt(z0 ** 2 + z1 ** 2 + z2 ** 2)
    norma = jnp.sqrt(y0[...] ** 2 + y1[...] ** 2 + y2[...] ** 2)
    dot = z0 * y0[...] + z1 * y1[...] + z2 * y2[...]
    cos = dot / (norma * normz)
    cosc = jnp.clip(cos, -1.0, 1.0)
    ang = _acos(cosc) * jnp.float32(57.29577951308232)
    msk = m[...]
    cnt = jnp.sum(msk)
    loss = jnp.sum(jnp.where(msk > 0.0, 1.0 - jnp.abs(cosc), 0.0)) / cnt
    angs = jnp.sum(jnp.where(msk > 0.0, ang, 0.0)) / cnt
    loss_ref[...] = jnp.reshape(loss, (1, 1))
    ang_ref[...] = jnp.reshape(angs, (1, 1))


def _finish(zparts, y, mask):
    n = y.shape[0]
    pad = _ROWS_PER_BLOCK * _LANES - n

    def prep(v):
        v = jnp.pad(v, (0, pad))
        return v.reshape(_ROWS_PER_BLOCK, _LANES)

    def prep2(v):
        return v.reshape(_ROWS_PER_BLOCK, _LANES)

    args = [prep2(zparts[0, i]) for i in range(3)]
    args += [prep2(zparts[1, i]) for i in range(3)]
    args += [prep(y[:, i].astype(jnp.float32)) for i in range(3)]
    args.append(prep(mask.astype(jnp.float32)))
    loss, ang = pl.pallas_call(
        _finish_body,
        out_shape=(
            jax.ShapeDtypeStruct((1, 1), jnp.float32),
            jax.ShapeDtypeStruct((1, 1), jnp.float32),
        ),
    )(*args)
    return loss[0, 0], ang[0, 0]


def kernel(out_scalar_field, x, y, edge_index, mask):
    E = edge_index.shape[1]
    ei = edge_index.reshape(2, E // _LANES, _LANES)
    a2, b2 = pl.pallas_call(
        _minmax_body,
        out_shape=(
            jax.ShapeDtypeStruct((E // _LANES, _LANES), jnp.int32),
            jax.ShapeDtypeStruct((E // _LANES, _LANES), jnp.int32),
        ),
    )(ei)
    a = a2.reshape(E)
    b = b2.reshape(E)
    n = out_scalar_field.shape[0]
    npad = _ROWS_PER_BLOCK * _LANES - n
    xf = x.astype(jnp.float32)
    px0 = jnp.pad(xf[:, 0], (0, npad))
    px1 = jnp.pad(xf[:, 1], (0, npad))
    px2 = jnp.pad(xf[:, 2], (0, npad))
    pss = jnp.pad(out_scalar_field.astype(jnp.float32), (0, npad))
    zz = jnp.zeros((n + npad,), jnp.float32)
    zflat = _sc_dedup_call(a, b, px0, px1, px2, pss, zz)
    zparts = zflat.reshape(2, 3, n + npad)
    return _finish(zparts, y, mask)


# speedup vs baseline: 15.1490x; 1.0284x over previous
import numpy as np

import jax
import jax.numpy as jnp
from jax import lax
from jax.experimental import pallas as pl
from jax.experimental.pallas import tpu as pltpu
from jax.experimental.pallas import tpu_sc as plsc

_LANES = 128
_ROWS_PER_BLOCK = 782

_H = 1 << 18
_CH = 512
_NSUB = _CH // 16


def _minmax_body(ei_ref, a_ref, b_ref):
    e = ei_ref[...]
    a_ref[...] = jnp.minimum(e[0], e[1])
    b_ref[...] = jnp.maximum(e[0], e[1])


def _sc_dedup_call(a_e, b_e, px0, px1, px2, pss, zz):
    E = a_e.shape[0]
    N = px0.shape[0]
    EW = E // 16
    RZ = N // 16
    nchunks_b = -(-EW // _CH)
    IDCAP = 52480

    mesh = plsc.VectorSubcoreMesh(core_axis_name="c", subcore_axis_name="s")

    c_h1a = np.uint32(2654435761)
    c_h1b = np.uint32(2246822519)
    c_h2a = np.uint32(3266489917)
    c_h2b = np.uint32(668265263)
    c_pa = np.uint32(0x7FEB352D)
    c_pb = np.uint32(0x846CA68B)
    c_msk = np.uint32(_H - 1)

    def body(a_hbm, b_hbm, hx0, hx1, hx2, hss, zz_hbm, out_hbm,
             tab, ps0, ps1, ps2, ps3, zs0, zs1, zs2,
             ids, abuf, bbuf, hbuf, idbuf, wbuf, wabuf, wbbuf,
             pa0, pa1, pa2, pa3, pb0, pb1, pb2, pb3,
             cr0, cr1, cr2,
             sem1, sem2, sem3):
        c = lax.axis_index("c")
        s = lax.axis_index("s")
        iota = lax.iota(jnp.int32, 16)
        one16 = jnp.ones((16,), jnp.int32)
        zero16 = jnp.zeros((16,), jnp.int32)

        def hashes(av, bv):
            au = av.astype(jnp.uint32)
            bu = bv.astype(jnp.uint32)
            h1 = (au * c_h1a) ^ (bu * c_h1b)
            h2 = ((au * c_h2a) ^ (bu * c_h2b)) | np.uint32(1)
            pm = au * c_pa + bu * c_pb
            par = lax.shift_right_logical(pm, np.uint32(31)).astype(jnp.int32)
            return h1, h2, par

        r0 = s * RZ
        nfull = RZ // _CH
        rtail = RZ - nfull * _CH
        segs = [(k * _CH, _CH) for k in range(nfull)]
        if rtail:
            segs.append((nfull * _CH, rtail))
        for hsrc, sdst in ((hx0, ps0), (hx1, ps1), (hx2, ps2), (hss, ps3),
                           (zz_hbm, zs0), (zz_hbm, zs1), (zz_hbm, zs2)):
            for o, ln in segs:
                pltpu.sync_copy(hsrc.at[pl.ds(r0 + o, ln)], pa0.at[pl.ds(0, ln)])
                pltpu.sync_copy(pa0.at[pl.ds(0, ln)], sdst.at[pl.ds(r0 + o, ln)])

        base0 = s * EW

        def b_chunk(i, off):
            cb = base0 + i * _CH
            da = pltpu.async_copy(a_hbm.at[pl.ds(cb, _CH)], abuf, sem1)
            db = pltpu.async_copy(b_hbm.at[pl.ds(cb, _CH)], bbuf, sem1)
            da.wait()
            db.wait()
            rem = EW - i * _CH
            for j in range(_NSUB):
                av = abuf[pl.ds(16 * j, 16)]
                bv = bbuf[pl.ds(16 * j, 16)]
                h1, h2, par = hashes(av, bv)
                valid = (iota + (16 * j)) < rem
                active = valid & (par == c)
                h = (h1 & c_msk).astype(jnp.int32)
                hbuf[pl.ds(16 * j, 16)] = jnp.where(active, h, _H + s)
                idv = cb + 16 * j + iota
                idbuf[pl.ds(16 * j, 16)] = jnp.where(active, idv, 0)
                offc = jnp.minimum(off, IDCAP - 144)
                mi = jnp.where(active, one16, zero16)
                excl = plsc.cumsum(mi) - mi
                tgt = jnp.where(active, offc + excl, IDCAP - 16 + iota)
                plsc.store_scatter(ids, [tgt], idv)
                off = jnp.minimum(off + jnp.sum(mi), IDCAP - 144)
            pltpu.sync_copy(idbuf, tab.at[hbuf])
            return off

        n0 = lax.fori_loop(0, nchunks_b, b_chunk, jnp.int32(0))
        for k in range(8):
            ids[pl.ds(n0 + 16 * k, 16)] = zero16

        def scatter_phase(r, n):
            ru = r.astype(jnp.uint32)
            nch = (n + (_CH - 1)) // _CH

            def chunk(i, _):
                ib = i * _CH
                da = pltpu.async_copy(a_hbm.at[ids.at[pl.ds(ib, _CH)]], abuf, sem1)
                db = pltpu.async_copy(b_hbm.at[ids.at[pl.ds(ib, _CH)]], bbuf, sem1)
                da.wait()
                db.wait()
                rem = n - ib
                for j in range(_NSUB):
                    av = abuf[pl.ds(16 * j, 16)]
                    bv = bbuf[pl.ds(16 * j, 16)]
                    h1, h2, _ = hashes(av, bv)
                    valid = (iota + (16 * j)) < rem
                    h = ((h1 + ru * h2) & c_msk).astype(jnp.int32)
                    hbuf[pl.ds(16 * j, 16)] = jnp.where(valid, h, _H + s)
                    idv = ids[pl.ds(ib + 16 * j, 16)]
                    idbuf[pl.ds(16 * j, 16)] = jnp.where(valid, idv, 0)
                pltpu.sync_copy(idbuf, tab.at[hbuf])
                return 0

            lax.fori_loop(0, nch, chunk, 0)

        def check_phase(r, n):
            ru = r.astype(jnp.uint32)
            nch = (n + (_CH - 1)) // _CH

            def chunk(i, woff):
                ib = i * _CH
                da = pltpu.async_copy(a_hbm.at[ids.at[pl.ds(ib, _CH)]], abuf, sem1)
                db = pltpu.async_copy(b_hbm.at[ids.at[pl.ds(ib, _CH)]], bbuf, sem1)
                da.wait()
                db.wait()
                dg = [
                    pltpu.async_copy(ps0.at[abuf], pa0, sem2),
                    pltpu.async_copy(ps1.at[abuf], pa1, sem2),
                    pltpu.async_copy(ps2.at[abuf], pa2, sem2),
                    pltpu.async_copy(ps3.at[abuf], pa3, sem2),
                    pltpu.async_copy(ps0.at[bbuf], pb0, sem2),
                    pltpu.async_copy(ps1.at[bbuf], pb1, sem2),
                    pltpu.async_copy(ps2.at[bbuf], pb2, sem2),
                    pltpu.async_copy(ps3.at[bbuf], pb3, sem2),
                ]
                rem = n - ib
                for j in range(_NSUB):
                    av = abuf[pl.ds(16 * j, 16)]
                    bv = bbuf[pl.ds(16 * j, 16)]
                    h1, h2, _ = hashes(av, bv)
                    valid = (iota + (16 * j)) < rem
                    h = ((h1 + ru * h2) & c_msk).astype(jnp.int32)
                    hbuf[pl.ds(16 * j, 16)] = jnp.where(valid, h, _H + s)
                pltpu.sync_copy(tab.at[hbuf], wbuf)
                dwa = pltpu.async_copy(a_hbm.at[wbuf], wabuf, sem3)
                dwb = pltpu.async_copy(b_hbm.at[wbuf], wbbuf, sem3)
                for d in dg:
                    d.wait()
                dwa.wait()
                dwb.wait()
                for j in range(_NSUB):
                    jj = pl.ds(16 * j, 16)
                    av = abuf[jj]
                    bv = bbuf[jj]
                    wv = wbuf[jj]
                    wav = wabuf[jj]
                    wbv = wbbuf[jj]
                    idv = ids[pl.ds(ib + 16 * j, 16)]
                    valid = (iota + (16 * j)) < (n - ib)
                    selfw = (wv == idv) & valid
                    pmatch = (wav == av) & (wbv == bv)
                    surv = valid & jnp.logical_not(selfw) & jnp.logical_not(pmatch)
                    keepf = jnp.where(selfw, jnp.float32(1.0), jnp.float32(0.0))
                    dsv = (pb3[jj] - pa3[jj]) * keepf
                    cr0[jj] = (pb0[jj] - pa0[jj]) * dsv
                    cr1[jj] = (pb1[jj] - pa1[jj]) * dsv
                    cr2[jj] = (pb2[jj] - pa2[jj]) * dsv
                    woffc = jnp.minimum(woff, IDCAP - 144)
                    mi = jnp.where(surv, one16, zero16)
                    excl = plsc.cumsum(mi) - mi
                    tgt = jnp.where(surv, woffc + excl, IDCAP - 16 + iota)
                    plsc.store_scatter(ids, [tgt], idv)
                    woff = jnp.minimum(woff + jnp.sum(mi), IDCAP - 144)
                dz = [
                    pltpu.async_copy(cr0, zs0.at[abuf], sem3, add=True),
                    pltpu.async_copy(cr1, zs1.at[abuf], sem3, add=True),
                    pltpu.async_copy(cr2, zs2.at[abuf], sem3, add=True),
                    pltpu.async_copy(cr0, zs0.at[bbuf], sem3, add=True),
                    pltpu.async_copy(cr1, zs1.at[bbuf], sem3, add=True),
                    pltpu.async_copy(cr2, zs2.at[bbuf], sem3, add=True),
                ]
                for d in dz:
                    d.wait()
                return woff

            nnew = lax.fori_loop(0, nch, chunk, jnp.int32(0))
            for k in range(8):
                ids[pl.ds(nnew + 16 * k, 16)] = zero16
            return nnew

        plsc.subcore_barrier()
        n1 = check_phase(jnp.int32(0), n0)
        plsc.subcore_barrier()

        def rbody(r, n):
            scatter_phase(r, n)
            plsc.subcore_barrier()
            n2 = check_phase(r, n)
            plsc.subcore_barrier()
            return n2

        lax.fori_loop(1, 13, rbody, n1)

        ob = c * (3 * N) + s * RZ
        for k2, zsrc in enumerate((zs0, zs1, zs2)):
            for o, ln in segs:
                pltpu.sync_copy(zsrc.at[pl.ds(r0 + o, ln)], pa0.at[pl.ds(0, ln)])
                pltpu.sync_copy(pa0.at[pl.ds(0, ln)],
                                out_hbm.at[pl.ds(ob + k2 * N + o, ln)])

    f = pl.kernel(
        body,
        out_type=jax.ShapeDtypeStruct((2 * 3 * N,), jnp.float32),
        mesh=mesh,
        compiler_params=pltpu.CompilerParams(needs_layout_passes=False),
        scratch_types=[
            pltpu.VMEM_SHARED((_H + 16,), jnp.int32),
            pltpu.VMEM_SHARED((N,), jnp.float32),
            pltpu.VMEM_SHARED((N,), jnp.float32),
            pltpu.VMEM_SHARED((N,), jnp.float32),
            pltpu.VMEM_SHARED((N,), jnp.float32),
            pltpu.VMEM_SHARED((N,), jnp.float32),
            pltpu.VMEM_SHARED((N,), jnp.float32),
            pltpu.VMEM_SHARED((N,), jnp.float32),
            pltpu.VMEM((IDCAP,), jnp.int32),
            pltpu.VMEM((_CH,), jnp.int32),
            pltpu.VMEM((_CH,), jnp.int32),
            pltpu.VMEM((_CH,), jnp.int32),
            pltpu.VMEM((_CH,), jnp.int32),
            pltpu.VMEM((_CH,), jnp.int32),
            pltpu.VMEM((_CH,), jnp.int32),
            pltpu.VMEM((_CH,), jnp.int32),
            pltpu.VMEM((_CH,), jnp.float32),
            pltpu.VMEM((_CH,), jnp.float32),
            pltpu.VMEM((_CH,), jnp.float32),
            pltpu.VMEM((_CH,), jnp.float32),
            pltpu.VMEM((_CH,), jnp.float32),
            pltpu.VMEM((_CH,), jnp.float32),
            pltpu.VMEM((_CH,), jnp.float32),
            pltpu.VMEM((_CH,), jnp.float32),
            pltpu.VMEM((_CH,), jnp.float32),
            pltpu.VMEM((_CH,), jnp.float32),
            pltpu.VMEM((_CH,), jnp.float32),
            pltpu.SemaphoreType.DMA,
            pltpu.SemaphoreType.DMA,
            pltpu.SemaphoreType.DMA,
        ],
    )
    return f(a_e, b_e, px0, px1, px2, pss, zz)


def _acos(x):
    t = jnp.abs(x)
    r = jnp.sqrt(jnp.maximum(1.0 - t, 0.0)) * (
        1.5707288 + t * (-0.2121144 + t * (0.0742610 + t * (-0.0187293)))
    )
    return jnp.where(x < 0.0, jnp.float32(3.14159265358979) - r, r)


def _finish_body(za0, za1, za2, zb0, zb1, zb2, y0, y1, y2, m,
                 loss_ref, ang_ref):
    z0 = za0[...] + zb0[...]
    z1 = za1[...] + zb1[...]
    z2 = za2[...] + zb2[...]
    normz = jnp.sqrt(z0 ** 2 + z1 ** 2 + z2 ** 2)
    norma = jnp.sqrt(y0[...] ** 2 + y1[...] ** 2 + y2[...] ** 2)
    dot = z0 * y0[...] + z1 * y1[...] + z2 * y2[...]
    cos = dot / (norma * normz)
    cosc = jnp.clip(cos, -1.0, 1.0)
    ang = _acos(cosc) * jnp.float32(57.29577951308232)
    msk = m[...]
    cnt = jnp.sum(msk)
    loss = jnp.sum(jnp.where(msk > 0.0, 1.0 - jnp.abs(cosc), 0.0)) / cnt
    angs = jnp.sum(jnp.where(msk > 0.0, ang, 0.0)) / cnt
    loss_ref[...] = jnp.reshape(loss, (1, 1))
    ang_ref[...] = jnp.reshape(angs, (1, 1))


def _finish(zparts, y, mask):
    n = y.shape[0]
    pad = _ROWS_PER_BLOCK * _LANES - n

    def prep(v):
        v = jnp.pad(v, (0, pad))
        return v.reshape(_ROWS_PER_BLOCK, _LANES)

    def prep2(v):
        return v.reshape(_ROWS_PER_BLOCK, _LANES)

    args = [prep2(zparts[0, i]) for i in range(3)]
    args += [prep2(zparts[1, i]) for i in range(3)]
    args += [prep(y[:, i].astype(jnp.float32)) for i in range(3)]
    args.append(prep(mask.astype(jnp.float32)))
    loss, ang = pl.pallas_call(
        _finish_body,
        out_shape=(
            jax.ShapeDtypeStruct((1, 1), jnp.float32),
            jax.ShapeDtypeStruct((1, 1), jnp.float32),
        ),
    )(*args)
    return loss[0, 0], ang[0, 0]


def kernel(out_scalar_field, x, y, edge_index, mask):
    E = edge_index.shape[1]
    ei = edge_index.reshape(2, E // _LANES, _LANES)
    a2, b2 = pl.pallas_call(
        _minmax_body,
        out_shape=(
            jax.ShapeDtypeStruct((E // _LANES, _LANES), jnp.int32),
            jax.ShapeDtypeStruct((E // _LANES, _LANES), jnp.int32),
        ),
    )(ei)
    a = a2.reshape(E)
    b = b2.reshape(E)
    n = out_scalar_field.shape[0]
    npad = _ROWS_PER_BLOCK * _LANES - n
    xf = x.astype(jnp.float32)
    px0 = jnp.pad(xf[:, 0], (0, npad))
    px1 = jnp.pad(xf[:, 1], (0, npad))
    px2 = jnp.pad(xf[:, 2], (0, npad))
    pss = jnp.pad(out_scalar_field.astype(jnp.float32), (0, npad))
    zz = jnp.zeros((n + npad,), jnp.float32)
    zflat = _sc_dedup_call(a, b, px0, px1, px2, pss, zz)
    zparts = zflat.reshape(2, 3, n + npad)
    return _finish(zparts, y, mask)
